# Initial kernel scaffold; baseline (speedup 1.0000x reference)
import jax
import jax.numpy as jnp
from jax.experimental import pallas as pl


def kernel(hidden_states, W_router, W_gate, W_up, W_down):
    raise NotImplementedError("write your pallas kernel here")


# speedup vs baseline: 1.8274x; 1.8274x over previous
import functools

import jax
import jax.numpy as jnp
from jax import lax
from jax.experimental import pallas as pl
from jax.experimental.pallas import tpu as pltpu
from jax.experimental.pallas import tpu_sc as plsc

T = 2048
H = 1024
E = 8
K = 2
F = 2048
BT = 256
NP = 6144
NB = NP // BT
CSB = 512


def _router_body(x_ref, wr_ref, pos_ref, w1_ref, w2_ref, pc_ref):
    x = x_ref[...]
    logits = jnp.dot(x, wr_ref[...], preferred_element_type=jnp.float32)
    m = jnp.max(logits, axis=1, keepdims=True)
    ex = jnp.exp(logits - m)
    probs = ex / jnp.sum(ex, axis=1, keepdims=True)

    ii = lax.broadcasted_iota(jnp.int32, (T, E), 1)
    m1 = jnp.max(probs, axis=1, keepdims=True)
    i1 = jnp.min(jnp.where(probs == m1, ii, E), axis=1, keepdims=True)
    pmask = jnp.where(ii == i1, -1.0, probs)
    m2 = jnp.max(pmask, axis=1, keepdims=True)
    i2 = jnp.min(jnp.where(pmask == m2, ii, E), axis=1, keepdims=True)
    s = m1 + m2
    w1_ref[...] = m1 / s
    w2_ref[...] = m2 / s

    oh1 = (ii == i1).astype(jnp.float32)
    oh2 = (ii == i2).astype(jnp.float32)
    oh = jnp.concatenate([oh1, oh2], axis=0)

    r = lax.broadcasted_iota(jnp.int32, (CSB, CSB), 0)
    c = lax.broadcasted_iota(jnp.int32, (CSB, CSB), 1)
    L = (r >= c).astype(jnp.float32)
    carry = jnp.zeros((1, E), jnp.float32)
    segs = []
    for b in range(2 * T // CSB):
        seg = lax.slice(oh, (b * CSB, 0), ((b + 1) * CSB, E))
        cs = jnp.dot(L, seg, preferred_element_type=jnp.float32) + carry
        segs.append(cs)
        carry = lax.slice(cs, (CSB - 1, 0), (CSB, E))
    cums = jnp.concatenate(segs, axis=0)

    counts = carry
    pci = ((counts.astype(jnp.int32) + BT - 1) // BT) * BT
    pc_ref[...] = pci

    e_all = jnp.concatenate([i1, i2], axis=0)
    iia = lax.broadcasted_iota(jnp.int32, (2 * T, E), 1)
    pcf = pci.astype(jnp.float32)
    base = jnp.sum(jnp.where(iia < e_all, pcf, 0.0), axis=1, keepdims=True)
    rank = jnp.sum(oh * cums, axis=1, keepdims=True) - 1.0
    pos_ref[...] = (rank + base).astype(jnp.int32)


def _run_router(flat, w_router):
    return pl.pallas_call(
        _router_body,
        out_shape=(
            jax.ShapeDtypeStruct((2 * T, 1), jnp.int32),
            jax.ShapeDtypeStruct((T, 1), jnp.float32),
            jax.ShapeDtypeStruct((T, 1), jnp.float32),
            jax.ShapeDtypeStruct((1, E), jnp.int32),
        ),
    )(flat, w_router)


def _make_sc_mesh():
    return plsc.VectorSubcoreMesh(core_axis_name="c", subcore_axis_name="s")


def _dispatch_sc(flat, pos):
    info = plsc.get_sparse_core_info()
    nw = info.num_cores * info.num_subcores
    per_w = 2 * T // nw
    cn = 64

    @functools.partial(
        pl.kernel,
        out_type=jax.ShapeDtypeStruct((NP, H), jnp.float32),
        mesh=_make_sc_mesh(),
        scratch_types=[
            pltpu.VMEM((cn,), jnp.int32),
            pltpu.VMEM((cn, H), jnp.float32),
            pltpu.SemaphoreType.DMA,
        ],
    )
    def k(flat_hbm, pos_hbm, xg_hbm, idx_v, buf_v, sem):
        wid = lax.axis_index("s") * info.num_cores + lax.axis_index("c")
        for cc in range(per_w // cn):
            i0 = wid * per_w + cc * cn
            base = lax.rem(i0, T)
            pltpu.sync_copy(pos_hbm.at[pl.ds(i0, cn)], idx_v)
            pltpu.sync_copy(flat_hbm.at[pl.ds(base, cn)], buf_v)
            pltpu.async_copy(buf_v, xg_hbm.at[idx_v], sem).wait()

    return k(flat, pos)


def _combine_gather_sc(y, pos):
    info = plsc.get_sparse_core_info()
    nw = info.num_cores * info.num_subcores
    per_w = T // nw
    cn = 32

    @functools.partial(
        pl.kernel,
        out_type=(
            jax.ShapeDtypeStruct((T, H), jnp.float32),
            jax.ShapeDtypeStruct((T, H), jnp.float32),
        ),
        mesh=_make_sc_mesh(),
        scratch_types=[
            pltpu.VMEM((cn,), jnp.int32),
            pltpu.VMEM((cn, H), jnp.float32),
            pltpu.SemaphoreType.DMA,
        ],
    )
    def k(y_hbm, pos_hbm, y1_hbm, y2_hbm, idx_v, buf_v, sem):
        wid = lax.axis_index("s") * info.num_cores + lax.axis_index("c")
        for half, out_hbm in ((0, y1_hbm), (1, y2_hbm)):
            for cc in range(per_w // cn):
                t0 = wid * per_w + cc * cn
                pltpu.sync_copy(pos_hbm.at[pl.ds(half * T + t0, cn)], idx_v)
                pltpu.async_copy(y_hbm.at[idx_v], buf_v, sem).wait()
                pltpu.sync_copy(buf_v, out_hbm.at[pl.ds(t0, cn)])

    return k(y, pos)


def _ffn_body(xblk_ref, wexp_ref, act_ref, xg_ref, wg_ref, wu_ref, wd_ref,
              y_ref):
    b = pl.program_id(0)

    @pl.when(act_ref[b] == 1)
    def _():
        x = xg_ref[...]
        g = jnp.dot(x, wg_ref[0], preferred_element_type=jnp.float32)
        u = jnp.dot(x, wu_ref[0], preferred_element_type=jnp.float32)
        a = (g / (1.0 + jnp.exp(-g))) * u
        y_ref[...] = jnp.dot(a, wd_ref[0], preferred_element_type=jnp.float32)


def _run_ffn(xg, w_gate, w_up, w_down, xblk, wexp, active):
    grid_spec = pltpu.PrefetchScalarGridSpec(
        num_scalar_prefetch=3,
        grid=(NB,),
        in_specs=[
            pl.BlockSpec((BT, H), lambda b, xblk, wexp, act: (xblk[b], 0)),
            pl.BlockSpec((1, H, F), lambda b, xblk, wexp, act: (wexp[b], 0, 0)),
            pl.BlockSpec((1, H, F), lambda b, xblk, wexp, act: (wexp[b], 0, 0)),
            pl.BlockSpec((1, F, H), lambda b, xblk, wexp, act: (wexp[b], 0, 0)),
        ],
        out_specs=pl.BlockSpec((BT, H), lambda b, xblk, wexp, act: (xblk[b], 0)),
    )
    return pl.pallas_call(
        _ffn_body,
        grid_spec=grid_spec,
        out_shape=jax.ShapeDtypeStruct((NP, H), jnp.float32),
    )(xblk, wexp, active, xg, w_gate, w_up, w_down)


def _combine_body(w1_ref, w2_ref, y1_ref, y2_ref, out_ref):
    out_ref[...] = w1_ref[...] * y1_ref[...] + w2_ref[...] * y2_ref[...]


def _run_combine(w1, w2, y1, y2):
    return pl.pallas_call(
        _combine_body,
        grid=(T // BT,),
        in_specs=[
            pl.BlockSpec((BT, 1), lambda i: (i, 0)),
            pl.BlockSpec((BT, 1), lambda i: (i, 0)),
            pl.BlockSpec((BT, H), lambda i: (i, 0)),
            pl.BlockSpec((BT, H), lambda i: (i, 0)),
        ],
        out_specs=pl.BlockSpec((BT, H), lambda i: (i, 0)),
        out_shape=jax.ShapeDtypeStruct((T, H), jnp.float32),
    )(w1, w2, y1, y2)


def kernel(hidden_states, W_router, W_gate, W_up, W_down):
    b, s, h = hidden_states.shape
    flat = hidden_states.reshape(-1, h)

    pos2d, w1, w2, pc = _run_router(flat, W_router)
    pos = pos2d.reshape(2 * T)

    pcv = pc.reshape(E)
    ends = jnp.cumsum(pcv)
    nb = jnp.sum(pcv) // BT
    bidx = jnp.arange(NB, dtype=jnp.int32)
    mb = jnp.minimum(bidx, nb - 1)
    wexp = jnp.sum((mb[:, None] * BT >= ends[None, :]).astype(jnp.int32),
                   axis=1)
    active = (bidx < nb).astype(jnp.int32)
    xblk = jnp.where(bidx < nb, bidx, nb - 1).astype(jnp.int32)

    xg = _dispatch_sc(flat, pos)
    y = _run_ffn(xg, W_gate, W_up, W_down, xblk, wexp, active)
    y1, y2 = _combine_gather_sc(y, pos)
    out = _run_combine(w1, w2, y1, y2)
    return out.reshape(b, s, h)
